# baseline (speedup 1.0000x reference)
import functools

import jax
import jax.numpy as jnp
from jax.experimental import pallas as pl
from jax.experimental.pallas import tpu as pltpu

BLK = 1024


def _add_body(labels_ref, ctx_ref, emb_ref, out_ref):
    out_ref[...] = ctx_ref[...] + emb_ref[...]


@jax.jit
def kernel(ctx_vec, labels, emb_weight):
    b, seq, d = ctx_vec.shape
    n, _ = emb_weight.shape
    grid = (b, seq // BLK)
    grid_spec = pltpu.PrefetchScalarGridSpec(
        num_scalar_prefetch=1,
        grid=grid,
        in_specs=[
            pl.BlockSpec((1, BLK, d), lambda i, j, labels: (i, j, 0)),
            pl.BlockSpec((1, 1, d), lambda i, j, labels: (labels[i], 0, 0)),
        ],
        out_specs=pl.BlockSpec((1, BLK, d), lambda i, j, labels: (i, j, 0)),
    )
    return pl.pallas_call(
        _add_body,
        grid_spec=grid_spec,
        out_shape=jax.ShapeDtypeStruct(ctx_vec.shape, ctx_vec.dtype),
    )(labels.astype(jnp.int32), ctx_vec, emb_weight.reshape(n, 1, d))
